# baseline (speedup 1.0000x reference)
import functools

import jax
import jax.numpy as jnp
from jax import lax
from jax.experimental import pallas as pl
from jax.experimental.pallas import tpu as pltpu
from jax.experimental.pallas import tpu_sc as plsc

_BATCH = 16384
_SEQ = 200
_DIM = 64
_TROWS = 201
_LP = 224
_NCH = _LP // 16
_PAIRS = _LP // 2
_OPR = _SEQ // 2
_NW = 32
_ROWS_PW = _BATCH // _NW
_R = 4
_STEPS = _ROWS_PW // _R
_SLOTS = _R * _PAIRS
_IDXN = _SLOTS + _PAIRS


def _x_copy(x_hbm, xb, sx, base):
    return pltpu.make_async_copy(x_hbm.at[pl.ds(base, _R)],
                                 xb.at[pl.ds(0, _R), pl.ds(0, _SEQ)], sx)


def _g_copies(t2_hbm, idx, gb, sg):
    return [
        pltpu.make_async_copy(
            t2_hbm.at[idx.at[pl.ds(r * _PAIRS, _PAIRS)]],
            gb.at[r],
            sg,
        )
        for r in range(_R)
    ]


def _o_copies(out_hbm, gb, so, base):
    return [
        pltpu.make_async_copy(gb.at[r, pl.ds(0, _OPR), pl.ds(0, 128)],
                              out_hbm.at[pl.ds((base + r) * _OPR, _OPR)],
                              so)
        for r in range(_R)
    ]


def _sc_body(x_hbm, t2_hbm, out_hbm,
             xb0, xb1, id0, id1, gb0, gb1,
             sx0, sx1, sg0, sg1, so0, so1):
    c = lax.axis_index("c")
    s = lax.axis_index("s")
    wid = s * 2 + c
    base0 = wid * _ROWS_PW

    xbs = [(xb0, sx0), (xb1, sx1)]
    ids = [id0, id1]
    gbs = [(gb0, sg0, so0), (gb1, sg1, so1)]

    zeros16 = jnp.zeros((16,), jnp.int32)
    for xb, _ in xbs:
        for r in range(_R):
            xb[r, pl.ds(192, 16)] = zeros16
            xb[r, pl.ds(208, 16)] = zeros16

    lane = lax.iota(jnp.int32, 16)
    par = lane % 2
    cbase = lane // 2
    scl = 1 + (1 - par) * (_TROWS - 1)
    dump = jnp.int32(_SLOTS)

    _x_copy(x_hbm, xb0, sx0, base0).start()

    def substep(it, p):
        q = 1 - p
        xb, sx = xbs[p]
        idx = ids[p]
        gb, sg, so = gbs[p]
        gbq, sgq, soq = gbs[q]
        base = base0 + it * _R
        _x_copy(x_hbm, xb, sx, base).wait()
        for r in range(_R):
            carry = jnp.int32(0)
            for i in range(_NCH):
                v = xb[r, pl.ds(i * 16, 16)]
                m = jnp.minimum(jnp.abs(v), 1)
                cs = jnp.cumsum(m)
                val = (carry + cs) * m * scl
                ci = cbase + (8 * i)
                live = ci + (r * _PAIRS)
                plsc.store_scatter(idx, [live + par * (dump - live)], val)
                plsc.addupdate_scatter(idx, [live + (1 - par) * (dump - live)],
                                       val)
                carry = carry + cs[15]
        @pl.when(it >= 2)
        def _():
            for cp in _o_copies(out_hbm, gb, so, base0 + (it - 2) * _R):
                cp.wait()
        for cp in _g_copies(t2_hbm, idx, gb, sg):
            cp.start()
        @pl.when(it + 1 < _STEPS)
        def _():
            _x_copy(x_hbm, xbs[q][0], xbs[q][1], base + _R).start()
        @pl.when(it >= 1)
        def _():
            for cp in _g_copies(t2_hbm, ids[q], gbq, sgq):
                cp.wait()
            for cp in _o_copies(out_hbm, gbq, soq, base0 + (it - 1) * _R):
                cp.start()

    def step2(i2, carry_none):
        substep(i2 * 2, 0)
        substep(i2 * 2 + 1, 1)
        return carry_none

    lax.fori_loop(0, _STEPS // 2, step2, None)

    last = _STEPS - 1
    gb, sg, so = gbs[1]
    for cp in _g_copies(t2_hbm, ids[1], gb, sg):
        cp.wait()
    for cp in _o_copies(out_hbm, gb, so, base0 + last * _R):
        cp.start()
    for cp in _o_copies(out_hbm, gbs[0][0], gbs[0][2],
                        base0 + (last - 1) * _R):
        cp.wait()
    for cp in _o_copies(out_hbm, gb, so, base0 + last * _R):
        cp.wait()


def kernel(x, table):
    t2 = jnp.concatenate(
        [jnp.broadcast_to(table[:, None, :], (_TROWS, _TROWS, _DIM)),
         jnp.broadcast_to(table[None, :, :], (_TROWS, _TROWS, _DIM))],
        axis=-1,
    ).reshape(_TROWS * _TROWS, 2 * _DIM)
    t2 = jnp.pad(t2, ((0, 7), (0, 0)))
    mesh = plsc.VectorSubcoreMesh(core_axis_name="c", subcore_axis_name="s")
    f = functools.partial(
        pl.kernel,
        mesh=mesh,
        compiler_params=pltpu.CompilerParams(use_tc_tiling_on_sc=False,
                                             needs_layout_passes=False),
        out_type=jax.ShapeDtypeStruct((_BATCH * _OPR, 128), jnp.float32),
        scratch_types=[
            pltpu.VMEM((_R, _LP), jnp.int32),
            pltpu.VMEM((_R, _LP), jnp.int32),
            pltpu.VMEM((_IDXN,), jnp.int32),
            pltpu.VMEM((_IDXN,), jnp.int32),
            pltpu.VMEM((_R, _PAIRS, 128), jnp.float32),
            pltpu.VMEM((_R, _PAIRS, 128), jnp.float32),
            pltpu.SemaphoreType.DMA,
            pltpu.SemaphoreType.DMA,
            pltpu.SemaphoreType.DMA,
            pltpu.SemaphoreType.DMA,
            pltpu.SemaphoreType.DMA,
            pltpu.SemaphoreType.DMA,
        ],
    )(_sc_body)
    return f(x, t2).reshape(_BATCH, _SEQ, _DIM)


# speedup vs baseline: 4.4364x; 4.4364x over previous
import functools

import jax
import jax.numpy as jnp
from jax import lax
from jax.experimental import pallas as pl
from jax.experimental.pallas import tpu as pltpu
from jax.experimental.pallas import tpu_sc as plsc

_BATCH = 16384
_SEQ = 200
_DIM = 64
_TROWS = 201
_LP = 224
_NCH = _LP // 16
_PAIRS = _LP // 2
_OPR = _SEQ // 2
_NW = 32
_ROWS_PW = _BATCH // _NW
_R = 4
_STEPS = _ROWS_PW // _R
_IDXN = _R * _LP


def _x_copy(x_hbm, xb, sx, base):
    return pltpu.make_async_copy(x_hbm.at[pl.ds(base, _R)],
                                 xb.at[pl.ds(0, _R), pl.ds(0, _SEQ)], sx)


def _g_copies(tbuf, idx, gbe, gbo, sg):
    cps = []
    for r in range(_R):
        for h, gbuf in ((0, gbe), (1, gbo)):
            cps.append(pltpu.make_async_copy(
                tbuf.at[idx.at[pl.ds(r * _LP + h * _PAIRS, _PAIRS)]],
                gbuf.at[r],
                sg,
            ))
    return cps


def _o_copies(out_hbm, gbe, gbo, so, base):
    cps = []
    for r in range(_R):
        for h, gbuf in ((0, gbe), (1, gbo)):
            cps.append(pltpu.make_async_copy(
                gbuf.at[r, pl.ds(0, _OPR), pl.ds(0, _DIM)],
                out_hbm.at[pl.ds((base + r) * _OPR, _OPR),
                           pl.ds(h * _DIM, _DIM)],
                so,
            ))
    return cps


def _sc_body(x_hbm, table_hbm, out_hbm,
             tbuf, xb0, xb1, id0, id1, ge0, go0, ge1, go1,
             semt, sx0, sx1, sg0, sg1, so0, so1):
    c = lax.axis_index("c")
    s = lax.axis_index("s")
    wid = s * 2 + c
    base0 = wid * _ROWS_PW

    xbs = [(xb0, sx0), (xb1, sx1)]
    ids = [id0, id1]
    gbs = [(ge0, go0, sg0, so0), (ge1, go1, sg1, so1)]

    @pl.when(s == 0)
    def _():
        pltpu.make_async_copy(table_hbm, tbuf, semt).start()

    zeros16 = jnp.zeros((16,), jnp.int32)
    for xb, _ in xbs:
        for r in range(_R):
            xb[r, pl.ds(192, 16)] = zeros16
            xb[r, pl.ds(208, 16)] = zeros16

    @pl.when(s == 0)
    def _():
        pltpu.make_async_copy(table_hbm, tbuf, semt).wait()
    plsc.subcore_barrier()

    lane = lax.iota(jnp.int32, 16)
    emap = (lane % 2) * _PAIRS + lane // 2

    _x_copy(x_hbm, xb0, sx0, base0).start()

    def substep(it, p):
        q = 1 - p
        xb, sx = xbs[p]
        idx = ids[p]
        gbe, gbo, sg, so = gbs[p]
        gbeq, gboq, sgq, soq = gbs[q]
        base = base0 + it * _R
        _x_copy(x_hbm, xb, sx, base).wait()
        for r in range(_R):
            carry = jnp.int32(0)
            for i in range(_NCH):
                v = xb[r, pl.ds(i * 16, 16)]
                m = jnp.minimum(jnp.abs(v), 1)
                cs = jnp.cumsum(m)
                plsc.store_scatter(idx, [emap + (r * _LP + 8 * i)],
                                   (carry + cs) * m)
                carry = carry + cs[15]
        @pl.when(it >= 2)
        def _():
            for cp in _o_copies(out_hbm, gbe, gbo, so,
                                base0 + (it - 2) * _R):
                cp.wait()
        for cp in _g_copies(tbuf, idx, gbe, gbo, sg):
            cp.start()
        @pl.when(it + 1 < _STEPS)
        def _():
            _x_copy(x_hbm, xbs[q][0], xbs[q][1], base + _R).start()
        @pl.when(it >= 1)
        def _():
            for cp in _g_copies(tbuf, ids[q], gbeq, gboq, sgq):
                cp.wait()
            for cp in _o_copies(out_hbm, gbeq, gboq, soq,
                                base0 + (it - 1) * _R):
                cp.start()

    def step2(i2, carry_none):
        substep(i2 * 2, 0)
        substep(i2 * 2 + 1, 1)
        return carry_none

    lax.fori_loop(0, _STEPS // 2, step2, None)

    last = _STEPS - 1
    gbe, gbo, sg, so = gbs[1]
    for cp in _g_copies(tbuf, ids[1], gbe, gbo, sg):
        cp.wait()
    for cp in _o_copies(out_hbm, gbe, gbo, so, base0 + last * _R):
        cp.start()
    for cp in _o_copies(out_hbm, gbs[0][0], gbs[0][1], gbs[0][3],
                        base0 + (last - 1) * _R):
        cp.wait()
    for cp in _o_copies(out_hbm, gbe, gbo, so, base0 + last * _R):
        cp.wait()


def kernel(x, table):
    mesh = plsc.VectorSubcoreMesh(core_axis_name="c", subcore_axis_name="s")
    f = functools.partial(
        pl.kernel,
        mesh=mesh,
        compiler_params=pltpu.CompilerParams(use_tc_tiling_on_sc=False,
                                             needs_layout_passes=False),
        out_type=jax.ShapeDtypeStruct((_BATCH * _OPR, 128), jnp.float32),
        scratch_types=[
            pltpu.VMEM_SHARED((_TROWS, _DIM), jnp.float32),
            pltpu.VMEM((_R, _LP), jnp.int32),
            pltpu.VMEM((_R, _LP), jnp.int32),
            pltpu.VMEM((_IDXN,), jnp.int32),
            pltpu.VMEM((_IDXN,), jnp.int32),
            pltpu.VMEM((_R, _PAIRS, _DIM), jnp.float32),
            pltpu.VMEM((_R, _PAIRS, _DIM), jnp.float32),
            pltpu.VMEM((_R, _PAIRS, _DIM), jnp.float32),
            pltpu.VMEM((_R, _PAIRS, _DIM), jnp.float32),
            pltpu.SemaphoreType.DMA,
            pltpu.SemaphoreType.DMA,
            pltpu.SemaphoreType.DMA,
            pltpu.SemaphoreType.DMA,
            pltpu.SemaphoreType.DMA,
            pltpu.SemaphoreType.DMA,
            pltpu.SemaphoreType.DMA,
        ],
    )(_sc_body)
    return f(x, table).reshape(_BATCH, _SEQ, _DIM)


# speedup vs baseline: 4.6441x; 1.0468x over previous
import functools

import jax
import jax.numpy as jnp
from jax import lax
from jax.experimental import pallas as pl
from jax.experimental.pallas import tpu as pltpu
from jax.experimental.pallas import tpu_sc as plsc

_BATCH = 16384
_SEQ = 200
_DIM = 64
_TROWS = 201
_TPAD = 208
_LP = 208
_NCH = 13
_PAIRS = _LP // 2
_OPR = _SEQ // 2
_NW = 32
_ROWS_PW = _BATCH // _NW
_R = 4
_STEPS = _ROWS_PW // _R
_IDXN = _R * _LP


def _x_copy(x_hbm, xb, sx, base):
    return pltpu.make_async_copy(x_hbm.at[pl.ds(2 * base, 2 * _R)], xb, sx)


def _g_copies(tbuf, idx, gbe, gbo, sg):
    cps = []
    for r in range(_R):
        for h, gbuf in ((0, gbe), (1, gbo)):
            cps.append(pltpu.make_async_copy(
                tbuf.at[idx.at[pl.ds(r * _LP + h * _PAIRS, _PAIRS)]],
                gbuf.at[r],
                sg,
            ))
    return cps


def _o_copies(out_hbm, gbe, gbo, so, base):
    cps = []
    for r in range(_R):
        for h, gbuf in ((0, gbe), (1, gbo)):
            cps.append(pltpu.make_async_copy(
                gbuf.at[r, pl.ds(0, _OPR), pl.ds(0, _DIM)],
                out_hbm.at[pl.ds((base + r) * _OPR, _OPR),
                           pl.ds(h * _DIM, _DIM)],
                so,
            ))
    return cps


def _sc_body(x_hbm, table_hbm, out_hbm,
             tbuf, xb0, xb1, id0, id1, ge0, go0, ge1, go1,
             semt, sx0, sx1, sg0, sg1, so0, so1):
    c = lax.axis_index("c")
    s = lax.axis_index("s")
    wid = s * 2 + c
    base0 = wid * _ROWS_PW

    xbs = [(xb0, sx0), (xb1, sx1)]
    ids = [id0, id1]
    gbs = [(ge0, go0, sg0, so0), (ge1, go1, sg1, so1)]

    tsrc = table_hbm.at[pl.ds(0, _TPAD), pl.ds(0, _DIM)]

    @pl.when(s == 0)
    def _():
        pltpu.make_async_copy(tsrc, tbuf, semt).start()

    @pl.when(s == 0)
    def _():
        pltpu.make_async_copy(tsrc, tbuf, semt).wait()
    plsc.subcore_barrier()

    lane = lax.iota(jnp.int32, 16)
    emap = (lane % 2) * _PAIRS + lane // 2

    _x_copy(x_hbm, xb0, sx0, base0).start()

    def substep(it, p):
        q = 1 - p
        xb, sx = xbs[p]
        idx = ids[p]
        gbe, gbo, sg, so = gbs[p]
        gbeq, gboq, sgq, soq = gbs[q]
        base = base0 + it * _R
        _x_copy(x_hbm, xb, sx, base).wait()
        for r in range(_R):
            carry = jnp.int32(0)
            for i in range(_NCH):
                row = 2 * r + (0 if i < 8 else 1)
                col = 16 * i if i < 8 else 16 * (i - 8)
                v = xb[row, pl.ds(col, 16)]
                m = jnp.minimum(jnp.abs(v), 1)
                cs = jnp.cumsum(m)
                plsc.store_scatter(idx, [emap + (r * _LP + 8 * i)],
                                   (carry + cs) * m)
                carry = carry + cs[15]
        @pl.when(it >= 2)
        def _():
            for cp in _o_copies(out_hbm, gbe, gbo, so,
                                base0 + (it - 2) * _R):
                cp.wait()
        for cp in _g_copies(tbuf, idx, gbe, gbo, sg):
            cp.start()
        @pl.when(it + 1 < _STEPS)
        def _():
            _x_copy(x_hbm, xbs[q][0], xbs[q][1], base + _R).start()
        @pl.when(it >= 1)
        def _():
            for cp in _g_copies(tbuf, ids[q], gbeq, gboq, sgq):
                cp.wait()
            for cp in _o_copies(out_hbm, gbeq, gboq, soq,
                                base0 + (it - 1) * _R):
                cp.start()

    def step2(i2, carry_none):
        substep(i2 * 2, 0)
        substep(i2 * 2 + 1, 1)
        return carry_none

    lax.fori_loop(0, _STEPS // 2, step2, None)

    last = _STEPS - 1
    gbe, gbo, sg, so = gbs[1]
    for cp in _g_copies(tbuf, ids[1], gbe, gbo, sg):
        cp.wait()
    for cp in _o_copies(out_hbm, gbe, gbo, so, base0 + last * _R):
        cp.start()
    for cp in _o_copies(out_hbm, gbs[0][0], gbs[0][1], gbs[0][3],
                        base0 + (last - 1) * _R):
        cp.wait()
    for cp in _o_copies(out_hbm, gbe, gbo, so, base0 + last * _R):
        cp.wait()


def kernel(x, table):
    x2 = jnp.pad(x, ((0, 0), (0, 56))).reshape(_BATCH * 2, 128)
    t2 = jnp.pad(table, ((0, _TPAD - _TROWS), (0, 128 - _DIM)))
    mesh = plsc.VectorSubcoreMesh(core_axis_name="c", subcore_axis_name="s")
    f = functools.partial(
        pl.kernel,
        mesh=mesh,
        compiler_params=pltpu.CompilerParams(use_tc_tiling_on_sc=False,
                                             needs_layout_passes=False),
        out_type=jax.ShapeDtypeStruct((_BATCH * _OPR, 128), jnp.float32),
        scratch_types=[
            pltpu.VMEM_SHARED((_TPAD, _DIM), jnp.float32),
            pltpu.VMEM((2 * _R, 128), jnp.int32),
            pltpu.VMEM((2 * _R, 128), jnp.int32),
            pltpu.VMEM((_IDXN,), jnp.int32),
            pltpu.VMEM((_IDXN,), jnp.int32),
            pltpu.VMEM((_R, _PAIRS, _DIM), jnp.float32),
            pltpu.VMEM((_R, _PAIRS, _DIM), jnp.float32),
            pltpu.VMEM((_R, _PAIRS, _DIM), jnp.float32),
            pltpu.VMEM((_R, _PAIRS, _DIM), jnp.float32),
            pltpu.SemaphoreType.DMA,
            pltpu.SemaphoreType.DMA,
            pltpu.SemaphoreType.DMA,
            pltpu.SemaphoreType.DMA,
            pltpu.SemaphoreType.DMA,
            pltpu.SemaphoreType.DMA,
            pltpu.SemaphoreType.DMA,
        ],
    )(_sc_body)
    return f(x2, t2).reshape(_BATCH, _SEQ, _DIM)


# speedup vs baseline: 4.6728x; 1.0062x over previous
import functools

import jax
import jax.numpy as jnp
from jax import lax
from jax.experimental import pallas as pl
from jax.experimental.pallas import tpu as pltpu
from jax.experimental.pallas import tpu_sc as plsc

_BATCH = 16384
_SEQ = 200
_DIM = 64
_TROWS = 201
_LP = 208
_NCH = _LP // 16
_NW = 32
_ROWS_PW = _BATCH // _NW
_R = 4
_STEPS = _ROWS_PW // _R
_IDXN = _R * _LP
_GCHUNK = 104


def _x_copy(x_hbm, xb, sx, base):
    return pltpu.make_async_copy(x_hbm.at[pl.ds(base, _R)],
                                 xb.at[pl.ds(0, _R), pl.ds(0, _SEQ)], sx)


def _g_copies(tbuf, idx, gb, sg):
    return [
        pltpu.make_async_copy(
            tbuf.at[idx.at[pl.ds((2 * r + h) * _GCHUNK, _GCHUNK)]],
            gb.at[r, pl.ds(h * _GCHUNK, _GCHUNK)],
            sg,
        )
        for r in range(_R)
        for h in range(2)
    ]


def _o_copy(out_hbm, gb, so, base):
    return pltpu.make_async_copy(gb.at[pl.ds(0, _R), pl.ds(0, _SEQ)],
                                 out_hbm.at[pl.ds(base, _R)], so)


def _sc_body(x_hbm, table_hbm, out_hbm,
             tbuf, xb0, xb1, id0, id1, gb0, gb1,
             semt, sx0, sx1, sg0, sg1, so0, so1):
    c = lax.axis_index("c")
    s = lax.axis_index("s")
    wid = s * 2 + c
    base0 = wid * _ROWS_PW

    xbs = [(xb0, sx0), (xb1, sx1)]
    ids = [id0, id1]
    gbs = [(gb0, sg0, so0), (gb1, sg1, so1)]

    @pl.when(s == 0)
    def _():
        pltpu.make_async_copy(table_hbm, tbuf, semt).start()

    zeros16 = jnp.zeros((16,), jnp.int32)
    for xb, _ in xbs:
        for r in range(_R):
            xb[r, pl.ds(192, 16)] = zeros16

    @pl.when(s == 0)
    def _():
        pltpu.make_async_copy(table_hbm, tbuf, semt).wait()
    plsc.subcore_barrier()

    _x_copy(x_hbm, xb0, sx0, base0).start()

    def substep(it, p):
        q = 1 - p
        xb, sx = xbs[p]
        idx = ids[p]
        gb, sg, so = gbs[p]
        gbq, sgq, soq = gbs[q]
        base = base0 + it * _R
        _x_copy(x_hbm, xb, sx, base).wait()
        for r in range(_R):
            carry = jnp.int32(0)
            for i in range(_NCH):
                v = xb[r, pl.ds(i * 16, 16)]
                m = jnp.minimum(jnp.abs(v), 1)
                cs = jnp.cumsum(m)
                idx[pl.ds(r * _LP + i * 16, 16)] = (carry + cs) * m
                carry = carry + cs[15]
        @pl.when(it >= 2)
        def _():
            _o_copy(out_hbm, gb, so, base0 + (it - 2) * _R).wait()
        for cp in _g_copies(tbuf, idx, gb, sg):
            cp.start()
        @pl.when(it + 1 < _STEPS)
        def _():
            _x_copy(x_hbm, xbs[q][0], xbs[q][1], base + _R).start()
        @pl.when(it >= 1)
        def _():
            for cp in _g_copies(tbuf, ids[q], gbq, sgq):
                cp.wait()
            _o_copy(out_hbm, gbq, soq, base0 + (it - 1) * _R).start()

    def step2(i2, carry_none):
        substep(i2 * 2, 0)
        substep(i2 * 2 + 1, 1)
        return carry_none

    lax.fori_loop(0, _STEPS // 2, step2, None)

    last = _STEPS - 1
    gb, sg, so = gbs[1]
    for cp in _g_copies(tbuf, ids[1], gb, sg):
        cp.wait()
    _o_copy(out_hbm, gb, so, base0 + last * _R).start()
    _o_copy(out_hbm, gbs[0][0], gbs[0][2], base0 + (last - 1) * _R).wait()
    _o_copy(out_hbm, gb, so, base0 + last * _R).wait()


def kernel(x, table):
    mesh = plsc.VectorSubcoreMesh(core_axis_name="c", subcore_axis_name="s")
    f = functools.partial(
        pl.kernel,
        mesh=mesh,
        compiler_params=pltpu.CompilerParams(use_tc_tiling_on_sc=False,
                                             needs_layout_passes=False),
        out_type=jax.ShapeDtypeStruct((_BATCH, _SEQ, _DIM), jnp.float32),
        scratch_types=[
            pltpu.VMEM_SHARED((_TROWS, _DIM), jnp.float32),
            pltpu.VMEM((_R, _LP), jnp.int32),
            pltpu.VMEM((_R, _LP), jnp.int32),
            pltpu.VMEM((_IDXN,), jnp.int32),
            pltpu.VMEM((_IDXN,), jnp.int32),
            pltpu.VMEM((_R, _LP, _DIM), jnp.float32),
            pltpu.VMEM((_R, _LP, _DIM), jnp.float32),
            pltpu.SemaphoreType.DMA,
            pltpu.SemaphoreType.DMA,
            pltpu.SemaphoreType.DMA,
            pltpu.SemaphoreType.DMA,
            pltpu.SemaphoreType.DMA,
            pltpu.SemaphoreType.DMA,
            pltpu.SemaphoreType.DMA,
        ],
    )(_sc_body)
    return f(x, table)
